# baseline (speedup 1.0000x reference)
import functools

import jax
import jax.numpy as jnp
from jax.experimental import pallas as pl
from jax.experimental.pallas import tpu as pltpu

TOP_K = 2
NUM_EXPERTS = 8
TOKEN_TILE = 2048


def _moe_dense_kernel(x_ref, wg_ref, bg_ref, we_ref, be_ref, out_ref,
                      web_ref):
    i = pl.program_id(0)

    @pl.when(i == 0)
    def _cast_weights():
        for e in range(NUM_EXPERTS):
            web_ref[e] = we_ref[e].astype(jnp.bfloat16)

    x = x_ref[...]
    scores = jnp.dot(x, wg_ref[...], preferred_element_type=jnp.float32)
    scores = scores + bg_ref[...][None, :]
    m = jnp.max(scores, axis=-1, keepdims=True)
    ex = jnp.exp(scores - m)
    probs = ex / jnp.sum(ex, axis=-1, keepdims=True)
    lane = jax.lax.broadcasted_iota(jnp.int32, probs.shape, 1)
    i1 = jnp.argmax(probs, axis=-1, keepdims=True)
    mask1 = lane == i1
    neg = jnp.where(mask1, -jnp.inf, probs)
    i2 = jnp.argmax(neg, axis=-1, keepdims=True)
    mask2 = lane == i2
    cw = jnp.where(mask1 | mask2, probs, 0.0)

    xb = x.astype(jnp.bfloat16)
    acc = jnp.dot(cw, be_ref[...], preferred_element_type=jnp.float32)
    for e in range(NUM_EXPERTS):
        y = jnp.dot(xb, web_ref[e], preferred_element_type=jnp.float32)
        acc = acc + cw[:, e:e + 1] * y
    out_ref[...] = acc


@jax.jit
def kernel(x, Wg, bg, We, be):
    T, D = x.shape
    E, _, O = We.shape
    grid = (T // TOKEN_TILE,)
    return pl.pallas_call(
        _moe_dense_kernel,
        grid=grid,
        in_specs=[
            pl.BlockSpec((TOKEN_TILE, D), lambda i: (i, 0)),
            pl.BlockSpec((D, E), lambda i: (0, 0)),
            pl.BlockSpec((E,), lambda i: (0,)),
            pl.BlockSpec((E, D, O), lambda i: (0, 0, 0)),
            pl.BlockSpec((E, O), lambda i: (0, 0)),
        ],
        out_specs=pl.BlockSpec((TOKEN_TILE, O), lambda i: (i, 0)),
        out_shape=jax.ShapeDtypeStruct((T, O), jnp.float32),
        scratch_shapes=[pltpu.VMEM((E, D, O), jnp.bfloat16)],
        compiler_params=pltpu.CompilerParams(
            dimension_semantics=("arbitrary",),
        ),
    )(x, Wg, bg, We, be)


# speedup vs baseline: 1.0329x; 1.0329x over previous
import functools

import jax
import jax.numpy as jnp
from jax.experimental import pallas as pl
from jax.experimental.pallas import tpu as pltpu

TOP_K = 2
NUM_EXPERTS = 8
TOKEN_TILE = 1024


def _moe_dense_kernel(x_ref, wg_ref, bg_ref, we_hbm, be_ref, out_ref,
                      web_ref, wf0, wf1, sem0, sem1):
    i = pl.program_id(0)
    bufs = [wf0, wf1]
    sems = [sem0, sem1]

    @pl.when(i == 0)
    def _start_dma():
        pltpu.make_async_copy(we_hbm.at[pl.ds(0, 1)], wf0, sem0).start()
        pltpu.make_async_copy(we_hbm.at[pl.ds(1, 1)], wf1, sem1).start()

    x = x_ref[...]
    scores = jnp.dot(x, wg_ref[...], preferred_element_type=jnp.float32)
    scores = scores + bg_ref[...][None, :]
    m = jnp.max(scores, axis=-1, keepdims=True)
    ex = jnp.exp(scores - m)
    probs = ex / jnp.sum(ex, axis=-1, keepdims=True)
    lane = jax.lax.broadcasted_iota(jnp.int32, probs.shape, 1)
    i1 = jnp.argmax(probs, axis=-1, keepdims=True)
    mask1 = lane == i1
    neg = jnp.where(mask1, -jnp.inf, probs)
    i2 = jnp.argmax(neg, axis=-1, keepdims=True)
    mask2 = lane == i2
    cw = jnp.where(mask1 | mask2, probs, 0.0)
    xb = x.astype(jnp.bfloat16)

    @pl.when(i == 0)
    def _first_tile():
        acc = jnp.dot(cw, be_ref[...], preferred_element_type=jnp.float32)
        for e in range(NUM_EXPERTS):
            b, s = bufs[e % 2], sems[e % 2]
            pltpu.make_async_copy(we_hbm.at[pl.ds(e, 1)], b, s).wait()
            web_ref[e] = b[0].astype(jnp.bfloat16)
            y = jnp.dot(xb, web_ref[e], preferred_element_type=jnp.float32)
            acc = acc + cw[:, e:e + 1] * y
            if e + 2 < NUM_EXPERTS:
                pltpu.make_async_copy(
                    we_hbm.at[pl.ds(e + 2, 1)], b, s).start()
        out_ref[...] = acc

    @pl.when(i > 0)
    def _later_tiles():
        acc = jnp.dot(cw, be_ref[...], preferred_element_type=jnp.float32)
        for e in range(NUM_EXPERTS):
            y = jnp.dot(xb, web_ref[e], preferred_element_type=jnp.float32)
            acc = acc + cw[:, e:e + 1] * y
        out_ref[...] = acc


@jax.jit
def kernel(x, Wg, bg, We, be):
    T, D = x.shape
    E, _, O = We.shape
    grid = (T // TOKEN_TILE,)
    return pl.pallas_call(
        _moe_dense_kernel,
        grid=grid,
        in_specs=[
            pl.BlockSpec((TOKEN_TILE, D), lambda i: (i, 0)),
            pl.BlockSpec((D, E), lambda i: (0, 0)),
            pl.BlockSpec((E,), lambda i: (0,)),
            pl.BlockSpec(memory_space=pltpu.MemorySpace.HBM),
            pl.BlockSpec((E, O), lambda i: (0, 0)),
        ],
        out_specs=pl.BlockSpec((TOKEN_TILE, O), lambda i: (i, 0)),
        out_shape=jax.ShapeDtypeStruct((T, O), jnp.float32),
        scratch_shapes=[
            pltpu.VMEM((E, D, O), jnp.bfloat16),
            pltpu.VMEM((1, D, O), jnp.float32),
            pltpu.VMEM((1, D, O), jnp.float32),
            pltpu.SemaphoreType.DMA,
            pltpu.SemaphoreType.DMA,
        ],
        compiler_params=pltpu.CompilerParams(
            dimension_semantics=("arbitrary",),
        ),
    )(x, Wg, bg, We, be)


# speedup vs baseline: 1.0333x; 1.0004x over previous
import functools

import jax
import jax.numpy as jnp
from jax.experimental import pallas as pl
from jax.experimental.pallas import tpu as pltpu

TOP_K = 2
NUM_EXPERTS = 8
TOKEN_TILE = 1024


def _moe_dense_kernel(x_ref, wg_ref, bg_ref, we_ref, be_ref, out_ref,
                      web_ref):
    i = pl.program_id(0)

    @pl.when(i == 0)
    def _cast_weights():
        for e in range(NUM_EXPERTS):
            web_ref[e] = we_ref[e].astype(jnp.bfloat16)

    x = x_ref[...]
    scores = jnp.dot(x, wg_ref[...], preferred_element_type=jnp.float32)
    scores = scores + bg_ref[...][None, :]
    m = jnp.max(scores, axis=-1, keepdims=True)
    ex = jnp.exp(scores - m)
    probs = ex / jnp.sum(ex, axis=-1, keepdims=True)
    lane = jax.lax.broadcasted_iota(jnp.int32, probs.shape, 1)
    i1 = jnp.argmax(probs, axis=-1, keepdims=True)
    mask1 = lane == i1
    neg = jnp.where(mask1, -jnp.inf, probs)
    i2 = jnp.argmax(neg, axis=-1, keepdims=True)
    mask2 = lane == i2
    cw = jnp.where(mask1 | mask2, probs, 0.0)

    xb = x.astype(jnp.bfloat16)
    acc = jnp.dot(cw, be_ref[...], preferred_element_type=jnp.float32)
    for e in range(NUM_EXPERTS):
        y = jnp.dot(xb, web_ref[e], preferred_element_type=jnp.float32)
        acc = acc + cw[:, e:e + 1] * y
    out_ref[...] = acc


@jax.jit
def kernel(x, Wg, bg, We, be):
    T, D = x.shape
    E, _, O = We.shape
    grid = (T // TOKEN_TILE,)
    return pl.pallas_call(
        _moe_dense_kernel,
        grid=grid,
        in_specs=[
            pl.BlockSpec((TOKEN_TILE, D), lambda i: (i, 0)),
            pl.BlockSpec((D, E), lambda i: (0, 0)),
            pl.BlockSpec((E,), lambda i: (0,)),
            pl.BlockSpec((E, D, O), lambda i: (0, 0, 0)),
            pl.BlockSpec((E, O), lambda i: (0, 0)),
        ],
        out_specs=pl.BlockSpec((TOKEN_TILE, O), lambda i: (i, 0)),
        out_shape=jax.ShapeDtypeStruct((T, O), jnp.float32),
        scratch_shapes=[pltpu.VMEM((E, D, O), jnp.bfloat16)],
        compiler_params=pltpu.CompilerParams(
            dimension_semantics=("arbitrary",),
        ),
    )(x, Wg, bg, We, be)
